# baseline (speedup 1.0000x reference)
import functools

import jax
import jax.numpy as jnp
from jax import lax
from jax.experimental import pallas as pl
from jax.experimental.pallas import tpu as pltpu
from jax.experimental.pallas import tpu_sc as plsc

_D = 64
_PAD = 0
_NC = 2
_NS = 16
_NW = _NC * _NS
_BB = 128


def _embed_lookup(text, table, b, s):
  mesh = plsc.VectorSubcoreMesh(core_axis_name="c", subcore_axis_name="s")
  iota16 = lambda: lax.iota(jnp.int32, 16)
  full16 = lambda x: jnp.zeros((16,), jnp.int32) + x

  @functools.partial(
      pl.kernel,
      out_type=jax.ShapeDtypeStruct((s, _D // 8, _NW, 8, _BB), jnp.float32),
      mesh=mesh,
      compiler_params=pltpu.CompilerParams(
          needs_layout_passes=False, use_tc_tiling_on_sc=False),
      scratch_types=[
          pltpu.VMEM((_BB, s), jnp.int32),
          pltpu.VMEM((s, _BB), jnp.int32),
          [pltpu.VMEM((_BB, _D), jnp.float32) for _ in range(4)],
          [pltpu.VMEM((8, 8, _BB), jnp.float32) for _ in range(2)],
          pltpu.VMEM((16,), jnp.int32),
          [pltpu.SemaphoreType.DMA for _ in range(4)],
          [pltpu.SemaphoreType.DMA for _ in range(2)],
      ],
  )
  def run(text_hbm, table_hbm, out_hbm, slab_v, idxT_v, rbuf, tbuf, flag_v,
          gsems, osems):
    wid = lax.axis_index("s") * _NC + lax.axis_index("c")

    pltpu.sync_copy(text_hbm.at[pl.ds(_BB * wid, _BB)], slab_v)
    for g in range(_BB // 16):
      rowv = iota16() + 16 * g

      def s_body(q, c, _rowv=rowv, _g=g):
        v = plsc.load_gather(slab_v.at[...], [_rowv, full16(q)])
        idxT_v[q, pl.ds(16 * _g, 16)] = v
        return c

      lax.fori_loop(0, s, s_body, 0)

    def fire(q, r):
      pltpu.make_async_copy(
          table_hbm.at[idxT_v.at[q]], rbuf[r], gsems[r]).start()

    def gwait(q, r):
      pltpu.make_async_copy(
          table_hbm.at[idxT_v.at[q]], rbuf[r], gsems[r]).wait()

    def out_start(q, ot):
      pltpu.make_async_copy(
          tbuf[ot], out_hbm.at[q, :, wid], osems[ot]).start()

    def out_wait(q, ot):
      pltpu.make_async_copy(
          tbuf[ot], out_hbm.at[q, :, wid], osems[ot]).wait()

    def fixup(q, r):
      idx_row = idxT_v.at[q]
      msk_acc = idx_row[pl.ds(0, 16)] == _PAD
      for g in range(1, _BB // 16):
        msk_acc = msk_acc | (idx_row[pl.ds(16 * g, 16)] == _PAD)
      flag_v[...] = jnp.zeros((16,), jnp.int32)
      plsc.store_scatter(flag_v.at[...], [jnp.zeros((16,), jnp.int32)],
                         jnp.ones((16,), jnp.int32), mask=msk_acc)
      nz = flag_v[...][0]

      @pl.when(nz != 0)
      def _fix():
        zero16 = jnp.zeros((16,), jnp.float32)
        for g in range(_BB // 16):
          v = idx_row[pl.ds(16 * g, 16)]
          msk = v == _PAD
          rowv = 16 * g + iota16()

          def cbody(c, carry, _rowv=rowv, _msk=msk):
            plsc.store_scatter(rbuf[r].at[...], [_rowv, full16(c)], zero16,
                               mask=_msk)
            return carry

          lax.fori_loop(0, _D, cbody, 0)

    def transpose(r, ot):
      for jo in range(8):
        def jr_body(jr, c, _jo=jo):
          j = 8 * _jo + jr
          for g in range(_BB // 16):
            v = plsc.load_gather(rbuf[r].at[...], [iota16() + 16 * g,
                                                   full16(j)])
            tbuf[ot][_jo, jr, pl.ds(16 * g, 16)] = v
          return c

        lax.fori_loop(0, 8, jr_body, 0)

    fire(0, 0)
    fire(1, 1)

    def body4(t, carry):
      for bslot in range(4):
        q = 4 * t + bslot
        ot = bslot % 2
        gwait(q, bslot)

        @pl.when(q + 2 < s)
        def _next(_q=q, _b=bslot):
          fire(_q + 2, (_b + 2) % 4)

        fixup(q, bslot)

        @pl.when(q >= 2)
        def _drain(_q=q, _ot=ot):
          out_wait(_q - 2, _ot)

        transpose(bslot, ot)
        out_start(q, ot)
      return carry

    lax.fori_loop(0, s // 4, body4, 0)
    out_wait(s - 2, 0)
    out_wait(s - 1, 1)

  return run(text, table)


def kernel(text, text_lengths, embedding_weight):
  del text_lengths
  b, s = text.shape
  assert b == _NW * _BB and s % 4 == 0
  out5d = _embed_lookup(text.astype(jnp.int32), embedding_weight, b, s)
  return out5d.transpose((2, 4, 0, 1, 3)).reshape(b, s, _D)
